# baseline (speedup 1.0000x reference)
import functools

import jax
import jax.numpy as jnp
from jax import lax
from jax.experimental import pallas as pl
from jax.experimental.pallas import tpu as pltpu
from jax.experimental.pallas import tpu_sc as plsc

N_NODES = 10000
N_EDGES = 320000
D = 128

NC, NS, LANES = 2, 16, 16
NW = NC * NS
CHUNK = 128
NCHUNK = 80
EPW = NCHUNK * CHUNK
E_PAD = NW * EPW
N_PAD = 10240
RPT = N_PAD // NS


def _agg_body(with_counts, *refs):
    if with_counts:
        (feat, src3, dst3, part_out, cnt_out,
         idx_s, idx_d, rows, ones, zv, acc, cnt) = refs
    else:
        (feat, src3, dst3, part_out,
         idx_s, idx_d, rows, acc) = refs

    cid = lax.axis_index("c")
    sid = lax.axis_index("s")
    wid = cid * NS + sid

    pltpu.sync_copy(src3.at[wid], idx_s)
    pltpu.sync_copy(dst3.at[wid], idx_d)

    def zrow(i, _):
        for j in range(D // LANES):
            rows[i, pl.ds(j * LANES, LANES)] = jnp.zeros((LANES,), jnp.float32)
        return 0
    lax.fori_loop(0, CHUNK, zrow, 0)
    base = sid * RPT
    for z in range(RPT // CHUNK):
        pltpu.sync_copy(rows.at[pl.ds(0, CHUNK)],
                        acc.at[pl.ds(base + z * CHUNK, CHUNK)])

    if with_counts:
        def zvec(i, _):
            zv[pl.ds(i * LANES, LANES)] = jnp.zeros((LANES,), jnp.float32)
            return 0
        lax.fori_loop(0, RPT // LANES, zvec, 0)
        pltpu.sync_copy(zv, cnt.at[pl.ds(base, RPT)])
        for j in range(CHUNK // LANES):
            ones[pl.ds(j * LANES, LANES)] = jnp.ones((LANES,), jnp.float32)

    plsc.subcore_barrier()

    def body(g, _):
        buf = rows.at[pl.ds(0, CHUNK)]
        pltpu.sync_copy(feat.at[idx_s.at[g]], buf)
        pltpu.sync_copy(buf, acc.at[idx_d.at[g]], add=True)
        if with_counts:
            pltpu.sync_copy(ones, cnt.at[idx_d.at[g]], add=True)
        return 0
    lax.fori_loop(0, NCHUNK, body, 0)

    plsc.subcore_barrier()

    pltpu.sync_copy(acc.at[pl.ds(base, RPT)], part_out.at[cid, pl.ds(base, RPT)])
    if with_counts:
        pltpu.sync_copy(cnt.at[pl.ds(base, RPT)], cnt_out.at[cid, pl.ds(base, RPT)])


def _make_agg(with_counts):
    out_type = [jax.ShapeDtypeStruct((NC, N_PAD, D), jnp.float32)]
    scratch = [
        pltpu.VMEM((NCHUNK, CHUNK), jnp.int32),
        pltpu.VMEM((NCHUNK, CHUNK), jnp.int32),
        pltpu.VMEM((CHUNK, D), jnp.float32),
    ]
    if with_counts:
        out_type.append(jax.ShapeDtypeStruct((NC, N_PAD), jnp.float32))
        scratch += [
            pltpu.VMEM((CHUNK,), jnp.float32),
            pltpu.VMEM((RPT,), jnp.float32),
        ]
    scratch.append(pltpu.VMEM_SHARED((N_PAD, D), jnp.float32))
    if with_counts:
        scratch.append(pltpu.VMEM_SHARED((N_PAD,), jnp.float32))
    mesh = plsc.VectorSubcoreMesh(
        core_axis_name="c", subcore_axis_name="s",
        num_cores=NC, num_subcores=NS)
    return pl.kernel(
        functools.partial(_agg_body, with_counts),
        out_type=out_type, mesh=mesh, scratch_types=scratch)


_agg_with_counts = _make_agg(True)
_agg_no_counts = _make_agg(False)


def _dense_body(relu, p0, p1, c_ref, x, wl, bl, wr, o_ref):
    cnt = jnp.maximum(c_ref[...], 1.0)
    mean = (p0[...] + p1[...]) / cnt
    y = (jnp.dot(mean, wl[...], preferred_element_type=jnp.float32)
         + bl[...]
         + jnp.dot(x[...], wr[...], preferred_element_type=jnp.float32))
    o_ref[...] = jnp.maximum(y, 0.0) if relu else y


_BLK = 400


def _dense(part, cvec, x, wl, bl, wr, relu):
    grid = (N_NODES // _BLK,)
    c_spec = pl.BlockSpec((_BLK, 1), lambda i: (i, 0))
    row_spec = pl.BlockSpec((_BLK, D), lambda i: (i, 0))
    w_spec = pl.BlockSpec((D, D), lambda i: (0, 0))
    b_spec = pl.BlockSpec((1, D), lambda i: (0, 0))
    return pl.pallas_call(
        functools.partial(_dense_body, relu),
        grid=grid,
        in_specs=[row_spec, row_spec, c_spec, row_spec, w_spec, b_spec, w_spec],
        out_specs=row_spec,
        out_shape=jax.ShapeDtypeStruct((N_NODES, D), jnp.float32),
    )(part[0, :N_NODES], part[1, :N_NODES], cvec, x, wl, bl, wr)


def kernel(x, edge_index, W_l1, b_l1, W_r1, W_l2, b_l2, W_r2):
    npad = E_PAD - N_EDGES
    src3 = jnp.concatenate(
        [edge_index[0], jnp.zeros((npad,), jnp.int32)]).reshape(NW, NCHUNK, CHUNK)
    pad_dst = N_NODES + (jnp.arange(npad, dtype=jnp.int32) % (N_PAD - N_NODES))
    dst3 = jnp.concatenate(
        [edge_index[1], pad_dst]).reshape(NW, NCHUNK, CHUNK)

    part1, cnt = _agg_with_counts(x, src3, dst3)
    cvec = (cnt[0, :N_NODES] + cnt[1, :N_NODES])[:, None]
    h = _dense(part1, cvec, x, W_l1, b_l1.reshape(1, D), W_r1, relu=True)

    (part2,) = _agg_no_counts(h, src3, dst3)
    out = _dense(part2, cvec, h, W_l2, b_l2.reshape(1, D), W_r2, relu=False)
    return out


# speedup vs baseline: 3.9427x; 3.9427x over previous
import functools

import jax
import jax.numpy as jnp
from jax import lax
from jax.experimental import pallas as pl
from jax.experimental.pallas import tpu as pltpu
from jax.experimental.pallas import tpu_sc as plsc

N_NODES = 10000
N_EDGES = 320000
D = 128

NC, NS, LANES = 2, 16, 16
NW = NC * NS
EPW = N_EDGES // NW
CHUNK = 125
NCHUNK = EPW // CHUNK
BLKC = 8
NBLK = NCHUNK // BLKC
RBUF = 128
N_PAD = 10240
RPT = N_PAD // NS


def _agg_body(with_counts, *refs):
    if with_counts:
        (feat, src3, dst3, part_out, cnt_out,
         srcv, dring, rows, ones, zv, acc, cnt, sem_i, sem_g0, sem_g1) = refs
    else:
        (feat, src3, dst3, part_out,
         srcv, dring, rows, acc, sem_i, sem_g0, sem_g1) = refs

    cid = lax.axis_index("c")
    sid = lax.axis_index("s")
    wid = cid * NS + sid

    pltpu.sync_copy(src3.at[wid], srcv)
    pltpu.async_copy(dst3.at[wid, pl.ds(0, BLKC)], dring.at[pl.ds(0, BLKC)],
                     sem_i)

    def zrow(i, _):
        for j in range(D // LANES):
            rows[i, pl.ds(j * LANES, LANES)] = jnp.zeros((LANES,), jnp.float32)
        return 0
    lax.fori_loop(0, RBUF, zrow, 0)
    base = sid * RPT
    for z in range(RPT // RBUF):
        pltpu.sync_copy(rows.at[pl.ds(0, RBUF)],
                        acc.at[pl.ds(base + z * RBUF, RBUF)])

    if with_counts:
        def zvec(i, _):
            zv[pl.ds(i * LANES, LANES)] = jnp.zeros((LANES,), jnp.float32)
            return 0
        lax.fori_loop(0, RPT // LANES, zvec, 0)
        pltpu.sync_copy(zv, cnt.at[pl.ds(base, RPT)])
        for j in range(RBUF // LANES):
            ones[pl.ds(j * LANES, LANES)] = jnp.ones((LANES,), jnp.float32)

    plsc.subcore_barrier()

    def gather(g, buf, sem):
        return pltpu.make_async_copy(
            feat.at[srcv.at[g]],
            rows.at[pl.ds(buf * RBUF, CHUNK)], sem)

    gather(0, 0, sem_g0).start()
    gather(1, 1, sem_g1).start()

    def step(g, buf, sem, slot, r):
        gather(g, buf, sem).wait()
        didx = dring.at[slot * BLKC + r]
        pltpu.sync_copy(rows.at[pl.ds(buf * RBUF, CHUNK)],
                        acc.at[didx], add=True)
        if with_counts:
            pltpu.sync_copy(ones.at[pl.ds(0, CHUNK)], cnt.at[didx], add=True)

        @pl.when(g + 2 < NCHUNK)
        def _():
            gather(g + 2, buf, sem).start()

    def pair(p, _):
        g0 = 2 * p
        m = p // (BLKC // 2)
        slot = lax.rem(m, 2)
        r0 = lax.rem(g0, BLKC)

        @pl.when(r0 == 0)
        def _():
            pltpu.make_async_copy(
                dst3.at[wid, pl.ds(m * BLKC, BLKC)],
                dring.at[pl.ds(slot * BLKC, BLKC)], sem_i).wait()

        step(g0, 0, sem_g0, slot, r0)
        step(g0 + 1, 1, sem_g1, slot, r0 + 1)

        @pl.when((r0 == BLKC // 2) & (m + 1 < NBLK))
        def _():
            pltpu.async_copy(
                dst3.at[wid, pl.ds((m + 1) * BLKC, BLKC)],
                dring.at[pl.ds(lax.rem(m + 1, 2) * BLKC, BLKC)], sem_i)
        return 0

    lax.fori_loop(0, NCHUNK // 2, pair, 0)

    plsc.subcore_barrier()

    pltpu.sync_copy(acc.at[pl.ds(base, RPT)], part_out.at[cid, pl.ds(base, RPT)])
    if with_counts:
        pltpu.sync_copy(cnt.at[pl.ds(base, RPT)], cnt_out.at[cid, pl.ds(base, RPT)])


def _make_agg(with_counts):
    out_type = [jax.ShapeDtypeStruct((NC, N_PAD, D), jnp.float32)]
    scratch = [
        pltpu.VMEM((NCHUNK, CHUNK), jnp.int32),
        pltpu.VMEM((2 * BLKC, CHUNK), jnp.int32),
        pltpu.VMEM((2 * RBUF, D), jnp.float32),
    ]
    if with_counts:
        out_type.append(jax.ShapeDtypeStruct((NC, N_PAD), jnp.float32))
        scratch += [
            pltpu.VMEM((RBUF,), jnp.float32),
            pltpu.VMEM((RPT,), jnp.float32),
        ]
    scratch.append(pltpu.VMEM_SHARED((N_PAD, D), jnp.float32))
    if with_counts:
        scratch.append(pltpu.VMEM_SHARED((N_PAD,), jnp.float32))
    scratch += [pltpu.SemaphoreType.DMA] * 3
    mesh = plsc.VectorSubcoreMesh(
        core_axis_name="c", subcore_axis_name="s",
        num_cores=NC, num_subcores=NS)
    return pl.kernel(
        functools.partial(_agg_body, with_counts),
        out_type=out_type, mesh=mesh, scratch_types=scratch)


_agg_with_counts = _make_agg(True)
_agg_no_counts = _make_agg(False)


def _dense_body(relu, p0, p1, c_ref, x, wl, bl, wr, o_ref):
    cnt = jnp.maximum(c_ref[...], 1.0)
    mean = (p0[...] + p1[...]) / cnt
    y = (jnp.dot(mean, wl[...], preferred_element_type=jnp.float32)
         + bl[...]
         + jnp.dot(x[...], wr[...], preferred_element_type=jnp.float32))
    o_ref[...] = jnp.maximum(y, 0.0) if relu else y


_BLK = 400


def _dense(part, cvec, x, wl, bl, wr, relu):
    grid = (N_NODES // _BLK,)
    c_spec = pl.BlockSpec((_BLK, 1), lambda i: (i, 0))
    row_spec = pl.BlockSpec((_BLK, D), lambda i: (i, 0))
    w_spec = pl.BlockSpec((D, D), lambda i: (0, 0))
    b_spec = pl.BlockSpec((1, D), lambda i: (0, 0))
    return pl.pallas_call(
        functools.partial(_dense_body, relu),
        grid=grid,
        in_specs=[row_spec, row_spec, c_spec, row_spec, w_spec, b_spec, w_spec],
        out_specs=row_spec,
        out_shape=jax.ShapeDtypeStruct((N_NODES, D), jnp.float32),
    )(part[0, :N_NODES], part[1, :N_NODES], cvec, x, wl, bl, wr)


def kernel(x, edge_index, W_l1, b_l1, W_r1, W_l2, b_l2, W_r2):
    src3 = edge_index[0].reshape(NW, NCHUNK, CHUNK)
    dst3 = edge_index[1].reshape(NW, NCHUNK, CHUNK)

    part1, cnt = _agg_with_counts(x, src3, dst3)
    cvec = (cnt[0, :N_NODES] + cnt[1, :N_NODES])[:, None]
    h = _dense(part1, cvec, x, W_l1, b_l1.reshape(1, D), W_r1, relu=True)

    (part2,) = _agg_no_counts(h, src3, dst3)
    out = _dense(part2, cvec, h, W_l2, b_l2.reshape(1, D), W_r2, relu=False)
    return out


# speedup vs baseline: 4.1205x; 1.0451x over previous
import functools

import jax
import jax.numpy as jnp
from jax import lax
from jax.experimental import pallas as pl
from jax.experimental.pallas import tpu as pltpu
from jax.experimental.pallas import tpu_sc as plsc

N_NODES = 10000
N_EDGES = 320000
D = 128

NC, NS, LANES = 2, 16, 16
NW = NC * NS
EPW = N_EDGES // NW
CHUNK = 125
NCHUNK = EPW // CHUNK
BLKC = 8
NBLK = NCHUNK // BLKC
RBUF = 128
N_PAD = 10240
RPT = N_PAD // NS


def _agg_body(with_counts, *refs):
    if with_counts:
        (feat, src3, dst3, part_out, cnt_out,
         srcv, dring, rows, ones, zv, acc, cnt, sem_i, sem_g0, sem_g1) = refs
    else:
        (feat, src3, dst3, part_out,
         srcv, dring, rows, acc, sem_i, sem_g0, sem_g1) = refs

    cid = lax.axis_index("c")
    sid = lax.axis_index("s")
    wid = cid * NS + sid

    pltpu.sync_copy(src3.at[wid], srcv)
    pltpu.async_copy(dst3.at[wid, pl.ds(0, BLKC)], dring.at[pl.ds(0, BLKC)],
                     sem_i)

    def zrow(i, _):
        for j in range(D // LANES):
            rows[i, pl.ds(j * LANES, LANES)] = jnp.zeros((LANES,), jnp.float32)
        return 0
    lax.fori_loop(0, RBUF, zrow, 0)
    base = sid * RPT
    for z in range(RPT // RBUF):
        pltpu.sync_copy(rows.at[pl.ds(0, RBUF)],
                        acc.at[pl.ds(base + z * RBUF, RBUF)])

    if with_counts:
        def zvec(i, _):
            zv[pl.ds(i * LANES, LANES)] = jnp.zeros((LANES,), jnp.float32)
            return 0
        lax.fori_loop(0, RPT // LANES, zvec, 0)
        pltpu.sync_copy(zv, cnt.at[pl.ds(base, RPT)])
        for j in range(RBUF // LANES):
            ones[pl.ds(j * LANES, LANES)] = jnp.ones((LANES,), jnp.float32)

    plsc.subcore_barrier()

    def gather(g, buf, sem):
        return pltpu.make_async_copy(
            feat.at[srcv.at[g]],
            rows.at[pl.ds(buf * RBUF, CHUNK)], sem)

    gather(0, 0, sem_g0).start()
    gather(1, 1, sem_g1).start()

    def step(g, buf, sem, slot, r):
        gather(g, buf, sem).wait()
        didx = dring.at[slot * BLKC + r]
        pltpu.sync_copy(rows.at[pl.ds(buf * RBUF, CHUNK)],
                        acc.at[didx], add=True)
        if with_counts:
            pltpu.sync_copy(ones.at[pl.ds(0, CHUNK)], cnt.at[didx], add=True)

        @pl.when(g + 2 < NCHUNK)
        def _():
            gather(g + 2, buf, sem).start()

    def pair(p, _):
        g0 = 2 * p
        m = p // (BLKC // 2)
        slot = lax.rem(m, 2)
        r0 = lax.rem(g0, BLKC)

        @pl.when(r0 == 0)
        def _():
            pltpu.make_async_copy(
                dst3.at[wid, pl.ds(m * BLKC, BLKC)],
                dring.at[pl.ds(slot * BLKC, BLKC)], sem_i).wait()

        step(g0, 0, sem_g0, slot, r0)
        step(g0 + 1, 1, sem_g1, slot, r0 + 1)

        @pl.when((r0 == BLKC // 2) & (m + 1 < NBLK))
        def _():
            pltpu.async_copy(
                dst3.at[wid, pl.ds((m + 1) * BLKC, BLKC)],
                dring.at[pl.ds(lax.rem(m + 1, 2) * BLKC, BLKC)], sem_i)
        return 0

    lax.fori_loop(0, NCHUNK // 2, pair, 0)

    plsc.subcore_barrier()

    pltpu.sync_copy(acc.at[pl.ds(base, RPT)], part_out.at[cid, pl.ds(base, RPT)])
    if with_counts:
        pltpu.sync_copy(cnt.at[pl.ds(base, RPT)], cnt_out.at[cid, pl.ds(base, RPT)])


def _make_agg(with_counts):
    out_type = [jax.ShapeDtypeStruct((NC, N_PAD, D), jnp.float32)]
    scratch = [
        pltpu.VMEM((NCHUNK, CHUNK), jnp.int32),
        pltpu.VMEM((2 * BLKC, CHUNK), jnp.int32),
        pltpu.VMEM((2 * RBUF, D), jnp.float32),
    ]
    if with_counts:
        out_type.append(jax.ShapeDtypeStruct((NC, N_PAD), jnp.float32))
        scratch += [
            pltpu.VMEM((RBUF,), jnp.float32),
            pltpu.VMEM((RPT,), jnp.float32),
        ]
    scratch.append(pltpu.VMEM_SHARED((N_PAD, D), jnp.float32))
    if with_counts:
        scratch.append(pltpu.VMEM_SHARED((N_PAD,), jnp.float32))
    scratch += [pltpu.SemaphoreType.DMA] * 3
    mesh = plsc.VectorSubcoreMesh(
        core_axis_name="c", subcore_axis_name="s",
        num_cores=NC, num_subcores=NS)
    return pl.kernel(
        functools.partial(_agg_body, with_counts),
        out_type=out_type, mesh=mesh, scratch_types=scratch)


_agg_with_counts = _make_agg(True)
_agg_no_counts = _make_agg(False)


def _dense_body(relu, p_ref, c_ref, x, wl, bl, wr, o_ref):
    cnt = jnp.maximum(c_ref[...], 1.0)
    mean = (p_ref[0] + p_ref[1]) / cnt
    y = (jnp.dot(mean, wl[...], preferred_element_type=jnp.float32)
         + bl[...]
         + jnp.dot(x[...], wr[...], preferred_element_type=jnp.float32))
    o_ref[...] = jnp.maximum(y, 0.0) if relu else y


_BLK = 400


def _dense(part, cvec, x, wl, bl, wr, relu):
    grid = (N_NODES // _BLK,)
    p_spec = pl.BlockSpec((NC, _BLK, D), lambda i: (0, i, 0))
    c_spec = pl.BlockSpec((_BLK, 1), lambda i: (i, 0))
    row_spec = pl.BlockSpec((_BLK, D), lambda i: (i, 0))
    w_spec = pl.BlockSpec((D, D), lambda i: (0, 0))
    b_spec = pl.BlockSpec((1, D), lambda i: (0, 0))
    return pl.pallas_call(
        functools.partial(_dense_body, relu),
        grid=grid,
        in_specs=[p_spec, c_spec, row_spec, w_spec, b_spec, w_spec],
        out_specs=row_spec,
        out_shape=jax.ShapeDtypeStruct((N_NODES, D), jnp.float32),
    )(part, cvec, x, wl, bl, wr)


def kernel(x, edge_index, W_l1, b_l1, W_r1, W_l2, b_l2, W_r2):
    src3 = edge_index[0].reshape(NW, NCHUNK, CHUNK)
    dst3 = edge_index[1].reshape(NW, NCHUNK, CHUNK)

    part1, cnt = _agg_with_counts(x, src3, dst3)
    cvec = (cnt[0, :N_NODES] + cnt[1, :N_NODES])[:, None]
    h = _dense(part1, cvec, x, W_l1, b_l1.reshape(1, D), W_r1, relu=True)

    (part2,) = _agg_no_counts(h, src3, dst3)
    out = _dense(part2, cvec, h, W_l2, b_l2.reshape(1, D), W_r2, relu=False)
    return out


# speedup vs baseline: 4.5357x; 1.1008x over previous
import functools

import jax
import jax.numpy as jnp
from jax import lax
from jax.experimental import pallas as pl
from jax.experimental.pallas import tpu as pltpu
from jax.experimental.pallas import tpu_sc as plsc

N_NODES = 10000
N_EDGES = 320000
D = 128

NC, NS, LANES = 2, 16, 16
NW = NC * NS
EPW = N_EDGES // NW
CHUNK = 125
NCHUNK = EPW // CHUNK
BLKC = 8
NBLK = NCHUNK // BLKC
RBUF = 128
N_PAD = 10240
RPT = N_PAD // NS


def _agg_body(with_counts, *refs):
    if with_counts:
        (feat, e4, part_out, cnt_out,
         srcv, dring, rows, ones, zv, acc, cnt, sem_i, sem_g0, sem_g1) = refs
    else:
        (feat, e4, part_out,
         srcv, dring, rows, acc, sem_i, sem_g0, sem_g1) = refs

    cid = lax.axis_index("c")
    sid = lax.axis_index("s")
    wid = cid * NS + sid

    pltpu.sync_copy(e4.at[0, wid], srcv)
    pltpu.async_copy(e4.at[1, wid, pl.ds(0, BLKC)], dring.at[pl.ds(0, BLKC)],
                     sem_i)

    def zrow(i, _):
        for j in range(D // LANES):
            rows[i, pl.ds(j * LANES, LANES)] = jnp.zeros((LANES,), jnp.float32)
        return 0
    lax.fori_loop(0, RBUF, zrow, 0)
    base = sid * RPT
    for z in range(RPT // RBUF):
        pltpu.sync_copy(rows.at[pl.ds(0, RBUF)],
                        acc.at[pl.ds(base + z * RBUF, RBUF)])

    if with_counts:
        def zvec(i, _):
            zv[pl.ds(i * LANES, LANES)] = jnp.zeros((LANES,), jnp.float32)
            return 0
        lax.fori_loop(0, RPT // LANES, zvec, 0)
        pltpu.sync_copy(zv, cnt.at[pl.ds(base, RPT)])
        for j in range(RBUF // LANES):
            ones[pl.ds(j * LANES, LANES)] = jnp.ones((LANES,), jnp.float32)

    plsc.subcore_barrier()

    def gather(g, buf, sem):
        return pltpu.make_async_copy(
            feat.at[srcv.at[g]],
            rows.at[pl.ds(buf * RBUF, CHUNK)], sem)

    gather(0, 0, sem_g0).start()
    gather(1, 1, sem_g1).start()

    def step(g, buf, sem, slot, r):
        gather(g, buf, sem).wait()
        didx = dring.at[slot * BLKC + r]
        pltpu.sync_copy(rows.at[pl.ds(buf * RBUF, CHUNK)],
                        acc.at[didx], add=True)
        if with_counts:
            pltpu.sync_copy(ones.at[pl.ds(0, CHUNK)], cnt.at[didx], add=True)

        @pl.when(g + 2 < NCHUNK)
        def _():
            gather(g + 2, buf, sem).start()

    def pair(p, _):
        g0 = 2 * p
        m = p // (BLKC // 2)
        slot = lax.rem(m, 2)
        r0 = lax.rem(g0, BLKC)

        @pl.when(r0 == 0)
        def _():
            pltpu.make_async_copy(
                e4.at[1, wid, pl.ds(m * BLKC, BLKC)],
                dring.at[pl.ds(slot * BLKC, BLKC)], sem_i).wait()

        step(g0, 0, sem_g0, slot, r0)
        step(g0 + 1, 1, sem_g1, slot, r0 + 1)

        @pl.when((r0 == BLKC // 2) & (m + 1 < NBLK))
        def _():
            pltpu.async_copy(
                e4.at[1, wid, pl.ds((m + 1) * BLKC, BLKC)],
                dring.at[pl.ds(lax.rem(m + 1, 2) * BLKC, BLKC)], sem_i)
        return 0

    lax.fori_loop(0, NCHUNK // 2, pair, 0)

    plsc.subcore_barrier()

    pltpu.sync_copy(acc.at[pl.ds(base, RPT)], part_out.at[cid, pl.ds(base, RPT)])
    if with_counts:
        pltpu.sync_copy(cnt.at[pl.ds(base, RPT)], cnt_out.at[cid, pl.ds(base, RPT)])


def _make_agg(with_counts):
    out_type = [jax.ShapeDtypeStruct((NC, N_PAD, D), jnp.float32)]
    scratch = [
        pltpu.VMEM((NCHUNK, CHUNK), jnp.int32),
        pltpu.VMEM((2 * BLKC, CHUNK), jnp.int32),
        pltpu.VMEM((2 * RBUF, D), jnp.float32),
    ]
    if with_counts:
        out_type.append(jax.ShapeDtypeStruct((NC, N_PAD), jnp.float32))
        scratch += [
            pltpu.VMEM((RBUF,), jnp.float32),
            pltpu.VMEM((RPT,), jnp.float32),
        ]
    scratch.append(pltpu.VMEM_SHARED((N_PAD, D), jnp.float32))
    if with_counts:
        scratch.append(pltpu.VMEM_SHARED((N_PAD,), jnp.float32))
    scratch += [pltpu.SemaphoreType.DMA] * 3
    mesh = plsc.VectorSubcoreMesh(
        core_axis_name="c", subcore_axis_name="s",
        num_cores=NC, num_subcores=NS)
    return pl.kernel(
        functools.partial(_agg_body, with_counts),
        out_type=out_type, mesh=mesh, scratch_types=scratch)


_agg_with_counts = _make_agg(True)
_agg_no_counts = _make_agg(False)


def _dense_body(relu, p_ref, c_ref, x, wl, bl, wr, o_ref):
    cnt = jnp.maximum(c_ref[...], 1.0)
    mean = (p_ref[0] + p_ref[1]) / cnt
    y = (jnp.dot(mean, wl[...], preferred_element_type=jnp.float32)
         + bl[...]
         + jnp.dot(x[...], wr[...], preferred_element_type=jnp.float32))
    o_ref[...] = jnp.maximum(y, 0.0) if relu else y


_BLK = 1000


def _dense(part, cvec, x, wl, bl, wr, relu):
    grid = (N_NODES // _BLK,)
    p_spec = pl.BlockSpec((NC, _BLK, D), lambda i: (0, i, 0))
    c_spec = pl.BlockSpec((_BLK, 1), lambda i: (i, 0))
    row_spec = pl.BlockSpec((_BLK, D), lambda i: (i, 0))
    w_spec = pl.BlockSpec((D, D), lambda i: (0, 0))
    b_spec = pl.BlockSpec((1, D), lambda i: (0, 0))
    return pl.pallas_call(
        functools.partial(_dense_body, relu),
        grid=grid,
        in_specs=[p_spec, c_spec, row_spec, w_spec, b_spec, w_spec],
        out_specs=row_spec,
        out_shape=jax.ShapeDtypeStruct((N_NODES, D), jnp.float32),
    )(part, cvec, x, wl, bl, wr)


def kernel(x, edge_index, W_l1, b_l1, W_r1, W_l2, b_l2, W_r2):
    e4 = edge_index.reshape(2, NW, NCHUNK, CHUNK)

    part1, cnt = _agg_with_counts(x, e4)
    cvec = (cnt[0, :N_NODES] + cnt[1, :N_NODES])[:, None]
    h = _dense(part1, cvec, x, W_l1, b_l1.reshape(1, D), W_r1, relu=True)

    (part2,) = _agg_no_counts(h, e4)
    out = _dense(part2, cvec, h, W_l2, b_l2.reshape(1, D), W_r2, relu=False)
    return out


# speedup vs baseline: 4.6236x; 1.0194x over previous
import functools

import jax
import jax.numpy as jnp
from jax import lax
from jax.experimental import pallas as pl
from jax.experimental.pallas import tpu as pltpu
from jax.experimental.pallas import tpu_sc as plsc

N_NODES = 10000
N_EDGES = 320000
D = 128

NC, NS, LANES = 2, 16, 16
NW = NC * NS
EPW = N_EDGES // NW
CHUNK = 125
NCHUNK = EPW // CHUNK
BLKC = 8
NBLK = NCHUNK // BLKC
RBUF = 128
N_PAD = 10240
RPT = N_PAD // NS


def _agg_body(with_counts, *refs):
    if with_counts:
        (feat, e4, part_out, cnt_out,
         srcv, dring, rows, ones, zv, acc, cnt, sem_i, sem_g0, sem_g1) = refs
    else:
        (feat, e4, part_out,
         srcv, dring, rows, acc, sem_i, sem_g0, sem_g1) = refs

    cid = lax.axis_index("c")
    sid = lax.axis_index("s")
    wid = cid * NS + sid

    pltpu.sync_copy(e4.at[0, wid], srcv)
    pltpu.async_copy(e4.at[1, wid, pl.ds(0, BLKC)], dring.at[pl.ds(0, BLKC)],
                     sem_i)

    def zrow(i, _):
        for j in range(D // LANES):
            rows[i, pl.ds(j * LANES, LANES)] = jnp.zeros((LANES,), jnp.float32)
        return 0
    lax.fori_loop(0, RBUF, zrow, 0)
    base = sid * RPT
    for z in range(RPT // RBUF):
        pltpu.sync_copy(rows.at[pl.ds(0, RBUF)],
                        acc.at[pl.ds(base + z * RBUF, RBUF)])

    if with_counts:
        def zvec(i, _):
            zv[pl.ds(i * LANES, LANES)] = jnp.zeros((LANES,), jnp.float32)
            return 0
        lax.fori_loop(0, RPT // LANES, zvec, 0)
        pltpu.sync_copy(zv, cnt.at[pl.ds(base, RPT)])
        for j in range(RBUF // LANES):
            ones[pl.ds(j * LANES, LANES)] = jnp.ones((LANES,), jnp.float32)

    plsc.subcore_barrier()

    def gather(g, buf, sem):
        return pltpu.make_async_copy(
            feat.at[srcv.at[g]],
            rows.at[pl.ds(buf * RBUF, CHUNK)], sem)

    gather(0, 0, sem_g0).start()
    gather(1, 1, sem_g1).start()

    def step(g, buf, sem, slot, r):
        gather(g, buf, sem).wait()
        didx = dring.at[slot * BLKC + r]
        pltpu.sync_copy(rows.at[pl.ds(buf * RBUF, CHUNK)],
                        acc.at[didx], add=True)
        if with_counts:
            pltpu.sync_copy(ones.at[pl.ds(0, CHUNK)], cnt.at[didx], add=True)

        @pl.when(g + 2 < NCHUNK)
        def _():
            gather(g + 2, buf, sem).start()

    def pair(p, _):
        g0 = 2 * p
        m = p // (BLKC // 2)
        slot = lax.rem(m, 2)
        r0 = lax.rem(g0, BLKC)

        @pl.when(r0 == 0)
        def _():
            pltpu.make_async_copy(
                e4.at[1, wid, pl.ds(m * BLKC, BLKC)],
                dring.at[pl.ds(slot * BLKC, BLKC)], sem_i).wait()

        step(g0, 0, sem_g0, slot, r0)
        step(g0 + 1, 1, sem_g1, slot, r0 + 1)

        @pl.when((r0 == BLKC // 2) & (m + 1 < NBLK))
        def _():
            pltpu.async_copy(
                e4.at[1, wid, pl.ds((m + 1) * BLKC, BLKC)],
                dring.at[pl.ds(lax.rem(m + 1, 2) * BLKC, BLKC)], sem_i)
        return 0

    lax.fori_loop(0, NCHUNK // 2, pair, 0)

    plsc.subcore_barrier()

    pltpu.sync_copy(acc.at[pl.ds(base, RPT)], part_out.at[cid, pl.ds(base, RPT)])
    if with_counts:
        pltpu.sync_copy(cnt.at[pl.ds(base, RPT)], cnt_out.at[cid, pl.ds(base, RPT)])


def _make_agg(with_counts):
    out_type = [jax.ShapeDtypeStruct((NC, N_PAD, D), jnp.float32)]
    scratch = [
        pltpu.VMEM((NCHUNK, CHUNK), jnp.int32),
        pltpu.VMEM((2 * BLKC, CHUNK), jnp.int32),
        pltpu.VMEM((2 * RBUF, D), jnp.float32),
    ]
    if with_counts:
        out_type.append(jax.ShapeDtypeStruct((NC, N_PAD), jnp.float32))
        scratch += [
            pltpu.VMEM((RBUF,), jnp.float32),
            pltpu.VMEM((RPT,), jnp.float32),
        ]
    scratch.append(pltpu.VMEM_SHARED((N_PAD, D), jnp.float32))
    if with_counts:
        scratch.append(pltpu.VMEM_SHARED((N_PAD,), jnp.float32))
    scratch += [pltpu.SemaphoreType.DMA] * 3
    mesh = plsc.VectorSubcoreMesh(
        core_axis_name="c", subcore_axis_name="s",
        num_cores=NC, num_subcores=NS)
    return pl.kernel(
        functools.partial(_agg_body, with_counts),
        out_type=out_type, mesh=mesh, scratch_types=scratch)


_agg_with_counts = _make_agg(True)
_agg_no_counts = _make_agg(False)


def _dense_body(relu, p_ref, c_ref, xr, wl, bl, o_ref):
    cnt = jnp.maximum(c_ref[...], 1.0)
    mean = (p_ref[0] + p_ref[1]) / cnt
    y = (jnp.dot(mean, wl[...], preferred_element_type=jnp.float32)
         + bl[...] + xr[...])
    o_ref[...] = jnp.maximum(y, 0.0) if relu else y


def _xr_body(x, wr, o_ref):
    o_ref[...] = jnp.dot(x[...], wr[...], preferred_element_type=jnp.float32)


_BLK = 2000


def _row_specs():
    row_spec = pl.BlockSpec((_BLK, D), lambda i: (i, 0))
    w_spec = pl.BlockSpec((D, D), lambda i: (0, 0))
    return row_spec, w_spec


def _xr(x, wr):
    row_spec, w_spec = _row_specs()
    return pl.pallas_call(
        _xr_body,
        grid=(N_NODES // _BLK,),
        in_specs=[row_spec, w_spec],
        out_specs=row_spec,
        out_shape=jax.ShapeDtypeStruct((N_NODES, D), jnp.float32),
    )(x, wr)


def _dense(part, cvec, xr, wl, bl, relu):
    row_spec, w_spec = _row_specs()
    p_spec = pl.BlockSpec((NC, _BLK, D), lambda i: (0, i, 0))
    c_spec = pl.BlockSpec((_BLK, 1), lambda i: (i, 0))
    b_spec = pl.BlockSpec((1, D), lambda i: (0, 0))
    return pl.pallas_call(
        functools.partial(_dense_body, relu),
        grid=(N_NODES // _BLK,),
        in_specs=[p_spec, c_spec, row_spec, w_spec, b_spec],
        out_specs=row_spec,
        out_shape=jax.ShapeDtypeStruct((N_NODES, D), jnp.float32),
    )(part, cvec, xr, wl, bl)


def kernel(x, edge_index, W_l1, b_l1, W_r1, W_l2, b_l2, W_r2):
    e4 = edge_index.reshape(2, NW, NCHUNK, CHUNK)

    xr1 = _xr(x, W_r1)
    part1, cnt = _agg_with_counts(x, e4)
    cvec = (cnt[0, :N_NODES] + cnt[1, :N_NODES])[:, None]
    h = _dense(part1, cvec, xr1, W_l1, b_l1.reshape(1, D), relu=True)

    xr2 = _xr(h, W_r2)
    (part2,) = _agg_no_counts(h, e4)
    out = _dense(part2, cvec, xr2, W_l2, b_l2.reshape(1, D), relu=False)
    return out


# speedup vs baseline: 4.6274x; 1.0008x over previous
import functools

import jax
import jax.numpy as jnp
from jax import lax
from jax.experimental import pallas as pl
from jax.experimental.pallas import tpu as pltpu
from jax.experimental.pallas import tpu_sc as plsc

N_NODES = 10000
N_EDGES = 320000
D = 128

NC, NS, LANES = 2, 16, 16
NW = NC * NS
EPW = N_EDGES // NW
CHUNK = 125
NCHUNK = EPW // CHUNK
BLKC = 8
NBLK = NCHUNK // BLKC
RBUF = 128
N_PAD = 10240
RPT = N_PAD // NS


def _agg_body(with_counts, *refs):
    if with_counts:
        (feat, e4, part_out, cnt_out,
         srcv, dring, rows, ones, zv, acc, cnt, sem_i, sem_g0, sem_g1) = refs
    else:
        (feat, e4, part_out,
         srcv, dring, rows, acc, sem_i, sem_g0, sem_g1) = refs

    cid = lax.axis_index("c")
    sid = lax.axis_index("s")
    wid = cid * NS + sid

    pltpu.sync_copy(e4.at[0, wid], srcv)
    pltpu.async_copy(e4.at[1, wid, pl.ds(0, BLKC)], dring.at[pl.ds(0, BLKC)],
                     sem_i)

    def zrow(i, _):
        for j in range(D // LANES):
            rows[i, pl.ds(j * LANES, LANES)] = jnp.zeros((LANES,), jnp.float32)
        return 0
    lax.fori_loop(0, RBUF, zrow, 0)
    base = sid * RPT
    for z in range(RPT // RBUF):
        pltpu.sync_copy(rows.at[pl.ds(0, RBUF)],
                        acc.at[pl.ds(base + z * RBUF, RBUF)])

    if with_counts:
        def zvec(i, _):
            zv[pl.ds(i * LANES, LANES)] = jnp.zeros((LANES,), jnp.float32)
            return 0
        lax.fori_loop(0, RPT // LANES, zvec, 0)
        pltpu.sync_copy(zv, cnt.at[pl.ds(base, RPT)])
        for j in range(RBUF // LANES):
            ones[pl.ds(j * LANES, LANES)] = jnp.ones((LANES,), jnp.float32)

    def gather(g, buf, sem):
        return pltpu.make_async_copy(
            feat.at[srcv.at[g]],
            rows.at[pl.ds(buf * RBUF, CHUNK)], sem)

    gather(0, 0, sem_g0).start()
    gather(1, 1, sem_g1).start()

    plsc.subcore_barrier()

    def step(g, buf, sem, slot, r):
        gather(g, buf, sem).wait()
        didx = dring.at[slot * BLKC + r]
        pltpu.sync_copy(rows.at[pl.ds(buf * RBUF, CHUNK)],
                        acc.at[didx], add=True)
        if with_counts:
            pltpu.sync_copy(ones.at[pl.ds(0, CHUNK)], cnt.at[didx], add=True)

        @pl.when(g + 2 < NCHUNK)
        def _():
            gather(g + 2, buf, sem).start()

    def pair(p, _):
        g0 = 2 * p
        m = p // (BLKC // 2)
        slot = lax.rem(m, 2)
        r0 = lax.rem(g0, BLKC)

        @pl.when(r0 == 0)
        def _():
            pltpu.make_async_copy(
                e4.at[1, wid, pl.ds(m * BLKC, BLKC)],
                dring.at[pl.ds(slot * BLKC, BLKC)], sem_i).wait()

        step(g0, 0, sem_g0, slot, r0)
        step(g0 + 1, 1, sem_g1, slot, r0 + 1)

        @pl.when((r0 == BLKC // 2) & (m + 1 < NBLK))
        def _():
            pltpu.async_copy(
                e4.at[1, wid, pl.ds((m + 1) * BLKC, BLKC)],
                dring.at[pl.ds(lax.rem(m + 1, 2) * BLKC, BLKC)], sem_i)
        return 0

    lax.fori_loop(0, NCHUNK // 2, pair, 0)

    plsc.subcore_barrier()

    pltpu.sync_copy(acc.at[pl.ds(base, RPT)], part_out.at[cid, pl.ds(base, RPT)])
    if with_counts:
        pltpu.sync_copy(cnt.at[pl.ds(base, RPT)], cnt_out.at[cid, pl.ds(base, RPT)])


def _make_agg(with_counts):
    out_type = [jax.ShapeDtypeStruct((NC, N_PAD, D), jnp.float32)]
    scratch = [
        pltpu.VMEM((NCHUNK, CHUNK), jnp.int32),
        pltpu.VMEM((2 * BLKC, CHUNK), jnp.int32),
        pltpu.VMEM((2 * RBUF, D), jnp.float32),
    ]
    if with_counts:
        out_type.append(jax.ShapeDtypeStruct((NC, N_PAD), jnp.float32))
        scratch += [
            pltpu.VMEM((RBUF,), jnp.float32),
            pltpu.VMEM((RPT,), jnp.float32),
        ]
    scratch.append(pltpu.VMEM_SHARED((N_PAD, D), jnp.float32))
    if with_counts:
        scratch.append(pltpu.VMEM_SHARED((N_PAD,), jnp.float32))
    scratch += [pltpu.SemaphoreType.DMA] * 3
    mesh = plsc.VectorSubcoreMesh(
        core_axis_name="c", subcore_axis_name="s",
        num_cores=NC, num_subcores=NS)
    return pl.kernel(
        functools.partial(_agg_body, with_counts),
        out_type=out_type, mesh=mesh, scratch_types=scratch)


_agg_with_counts = _make_agg(True)
_agg_no_counts = _make_agg(False)


def _dense_body(relu, p_ref, c_ref, xr, wl, bl, o_ref):
    cnt = jnp.maximum(c_ref[...], 1.0)
    mean = (p_ref[0] + p_ref[1]) / cnt
    y = (jnp.dot(mean, wl[...], preferred_element_type=jnp.float32)
         + bl[...] + xr[...])
    o_ref[...] = jnp.maximum(y, 0.0) if relu else y


def _xr_body(x, wr, o_ref):
    o_ref[...] = jnp.dot(x[...], wr[...], preferred_element_type=jnp.float32)


_BLK = 2000


def _row_specs():
    row_spec = pl.BlockSpec((_BLK, D), lambda i: (i, 0))
    w_spec = pl.BlockSpec((D, D), lambda i: (0, 0))
    return row_spec, w_spec


def _xr(x, wr):
    row_spec, w_spec = _row_specs()
    return pl.pallas_call(
        _xr_body,
        grid=(N_NODES // _BLK,),
        in_specs=[row_spec, w_spec],
        out_specs=row_spec,
        out_shape=jax.ShapeDtypeStruct((N_NODES, D), jnp.float32),
    )(x, wr)


def _dense(part, cvec, xr, wl, bl, relu):
    row_spec, w_spec = _row_specs()
    p_spec = pl.BlockSpec((NC, _BLK, D), lambda i: (0, i, 0))
    c_spec = pl.BlockSpec((_BLK, 1), lambda i: (i, 0))
    b_spec = pl.BlockSpec((1, D), lambda i: (0, 0))
    return pl.pallas_call(
        functools.partial(_dense_body, relu),
        grid=(N_NODES // _BLK,),
        in_specs=[p_spec, c_spec, row_spec, w_spec, b_spec],
        out_specs=row_spec,
        out_shape=jax.ShapeDtypeStruct((N_NODES, D), jnp.float32),
    )(part, cvec, xr, wl, bl)


def kernel(x, edge_index, W_l1, b_l1, W_r1, W_l2, b_l2, W_r2):
    e4 = edge_index.reshape(2, NW, NCHUNK, CHUNK)

    xr1 = _xr(x, W_r1)
    part1, cnt = _agg_with_counts(x, e4)
    cvec = (cnt[0, :N_NODES] + cnt[1, :N_NODES])[:, None]
    h = _dense(part1, cvec, xr1, W_l1, b_l1.reshape(1, D), relu=True)

    xr2 = _xr(h, W_r2)
    (part2,) = _agg_no_counts(h, e4)
    out = _dense(part2, cvec, xr2, W_l2, b_l2.reshape(1, D), relu=False)
    return out
